# Initial kernel scaffold; baseline (speedup 1.0000x reference)
import jax
import jax.numpy as jnp
from jax.experimental import pallas as pl


def kernel(x, W, b, predicate_matrix):
    raise NotImplementedError("write your pallas kernel here")


# speedup vs baseline: 12.6696x; 12.6696x over previous
import jax
import jax.numpy as jnp
from jax.experimental import pallas as pl


def _fused_kernel(x_ref, w_ref, b_ref, p_ref, o_ref):
    feats = jnp.dot(x_ref[...], w_ref[...], preferred_element_type=jnp.float32)
    feats = feats + b_ref[...]
    f = (feats > 0.5).astype(jnp.float32)
    fp = jax.lax.dot_general(
        f, p_ref[...], (((1,), (1,)), ((), ())),
        preferred_element_type=jnp.float32)
    o_ref[...] = fp - jnp.sum(f, axis=1, keepdims=True)


def kernel(x, W, b, predicate_matrix):
    bsz, d_in = x.shape
    nf = W.shape[1]
    nc = predicate_matrix.shape[0]
    bm = 256
    b2 = b.reshape(1, nf)
    return pl.pallas_call(
        _fused_kernel,
        grid=(bsz // bm,),
        in_specs=[
            pl.BlockSpec((bm, d_in), lambda i: (i, 0)),
            pl.BlockSpec((d_in, nf), lambda i: (0, 0)),
            pl.BlockSpec((1, nf), lambda i: (0, 0)),
            pl.BlockSpec((nc, nf), lambda i: (0, 0)),
        ],
        out_specs=pl.BlockSpec((bm, nc), lambda i: (i, 0)),
        out_shape=jax.ShapeDtypeStruct((bsz, nc), jnp.float32),
    )(x, W, b2, predicate_matrix)
